# baseline (speedup 1.0000x reference)
import functools

import jax
import jax.numpy as jnp
from jax import lax
from jax.experimental import pallas as pl
from jax.experimental.pallas import tpu as pltpu
from jax.experimental.pallas import tpu_sc as plsc

N = 10000
E = 320000
D = 128
OUT = 16
HEADS = 10
HID = OUT * HEADS
G = 64

RB = 1000
NBLK = N // RB
NW = 32
EPW = E // NW
HHID = HID // 2
EPW1 = E // 16
K1 = 50
CH1 = EPW1 // K1
K2 = 40
CH2 = EPW1 // K2
N2 = 10240
RPS = N2 // 16


def _mm2_body(x_ref, wla_ref, wlb_ref, wra_ref, wrb_ref, xl_ref, xr_ref):
    xb = x_ref[...]
    xl_ref[0] = jnp.dot(xb, wla_ref[...], preferred_element_type=jnp.float32)
    xl_ref[1] = jnp.dot(xb, wlb_ref[...], preferred_element_type=jnp.float32)
    xr_ref[0] = jnp.dot(xb, wra_ref[...], preferred_element_type=jnp.float32)
    xr_ref[1] = jnp.dot(xb, wrb_ref[...], preferred_element_type=jnp.float32)


def _mm2(x, wla, wlb, wra, wrb):
    wspec = pl.BlockSpec((D, HHID), lambda i: (0, 0))
    ospec = pl.BlockSpec((2, RB, HHID), lambda i: (0, i, 0))
    return pl.pallas_call(
        _mm2_body,
        grid=(NBLK,),
        in_specs=[pl.BlockSpec((RB, D), lambda i: (i, 0)),
                  wspec, wspec, wspec, wspec],
        out_specs=[ospec, ospec],
        out_shape=[
            jax.ShapeDtypeStruct((2, N, HHID), jnp.float32),
            jax.ShapeDtypeStruct((2, N, HHID), jnp.float32),
        ],
    )(x, wla, wlb, wra, wrb)


def _sc_mesh():
    return plsc.VectorSubcoreMesh(core_axis_name="c", subcore_axis_name="s",
                                  num_cores=2, num_subcores=16)


NBUF1 = 5
NBUF2 = 10


def _pass1_body(xl_hbm, xr_hbm, src_hbm, dst_hbm, att_hbm, z80_hbm, z16_hbm,
                u_out, den_out,
                srcg, dstg,
                xlb0, xlb1, xlb2, xlb3, xlb4,
                xrb0, xrb1, xrb2, xrb3, xrb4,
                wb0, wb1, wb2, wb3, wb4, attb, usp, densp,
                g0, g1, g2, g3, g4, s0, s1, s2, s3, s4):
    xlbs = [xlb0, xlb1, xlb2, xlb3, xlb4]
    xrbs = [xrb0, xrb1, xrb2, xrb3, xrb4]
    wbs = [wb0, wb1, wb2, wb3, wb4]
    gsems = [g0, g1, g2, g3, g4]
    ssems = [s0, s1, s2, s3, s4]
    c = lax.axis_index("c")
    s = lax.axis_index("s")
    rows = pl.ds(s * RPS, RPS)
    pltpu.sync_copy(z80_hbm, usp.at[rows])
    pltpu.sync_copy(z16_hbm, densp.at[rows])
    pltpu.sync_copy(att_hbm.at[c], attb)
    plsc.subcore_barrier()
    attvs = [attb[pl.ds(16 * h, 16)] for h in range(HEADS // 2)]

    def group(g, carry):
        c0 = g * NBUF1
        pltpu.sync_copy(src_hbm.at[s, pl.ds(c0, NBUF1)], srcg)
        pltpu.sync_copy(dst_hbm.at[s, pl.ds(c0, NBUF1)], dstg)
        cps = []
        for bi in range(NBUF1):
            cps.append(pltpu.async_copy(
                xl_hbm.at[c].at[srcg.at[bi]], xlbs[bi], gsems[bi]))
            cps.append(pltpu.async_copy(
                xr_hbm.at[c].at[dstg.at[bi]], xrbs[bi], gsems[bi]))
        scs = []
        for bi in range(NBUF1):
            xlb, xrb, wb = xlbs[bi], xrbs[bi], wbs[bi]
            cps[2 * bi].wait()
            cps[2 * bi + 1].wait()

            @plsc.parallel_loop(0, K1, unroll=3)
            def edge(e):
                lanei = lax.iota(jnp.int32, 16)
                xh = [xlb[e, pl.ds(OUT * h, OUT)] for h in range(HEADS // 2)]
                logitv = jnp.zeros((16,), jnp.float32)
                for h in range(HEADS // 2):
                    z = xh[h] + xrb[e, pl.ds(OUT * h, OUT)]
                    lz = jnp.maximum(z, 0.2 * z)
                    sh = jnp.sum(lz * attvs[h])
                    logitv = jnp.where(lanei == h, sh, logitv)
                w = jnp.exp(logitv)
                wb[e, :] = w
                for h in range(HEADS // 2):
                    xlb[e, pl.ds(OUT * h, OUT)] = xh[h] * w[h]
            scs.append(pltpu.async_copy(
                xlb, usp.at[dstg.at[bi]], gsems[bi], add=True))
            scs.append(pltpu.async_copy(
                wb, densp.at[dstg.at[bi]], ssems[bi], add=True))
        for cp in scs:
            cp.wait()
        return carry

    lax.fori_loop(0, CH1 // NBUF1, group, 0)
    plsc.subcore_barrier()
    pltpu.sync_copy(usp.at[rows], u_out.at[c, rows])
    pltpu.sync_copy(densp.at[rows], den_out.at[c, rows])


@functools.lru_cache(maxsize=None)
def _pass1_kernel():
    return pl.kernel(
        _pass1_body,
        out_type=[
            jax.ShapeDtypeStruct((2, N2, HHID), jnp.float32),
            jax.ShapeDtypeStruct((2, N2, 16), jnp.float32),
        ],
        mesh=_sc_mesh(),
        compiler_params=pltpu.CompilerParams(use_tc_tiling_on_sc=False,
                                             needs_layout_passes=False),
        scratch_types=(
            [pltpu.VMEM((NBUF1, K1), jnp.int32)] * 2
            + [pltpu.VMEM((K1, HHID), jnp.float32)] * (2 * NBUF1)
            + [pltpu.VMEM((K1, 16), jnp.float32)] * NBUF1
            + [pltpu.VMEM((HHID,), jnp.float32)]
            + [pltpu.VMEM_SHARED((N2, HHID), jnp.float32),
               pltpu.VMEM_SHARED((N2, 16), jnp.float32)]
            + [pltpu.SemaphoreType.DMA] * (2 * NBUF1)
        ),
    )


def _pass2_body(hs_hbm, src_hbm, dst_hbm, z80_hbm,
                a_out,
                srcg, dstg, rb0, rb1, rb2, rb3, rb4, rb5, rb6, rb7, rb8, rb9,
                asp,
                g0, g1, g2, g3, g4, g5, g6, g7, g8, g9,
                s0, s1, s2, s3, s4, s5, s6, s7, s8, s9):
    rowbs = [rb0, rb1, rb2, rb3, rb4, rb5, rb6, rb7, rb8, rb9]
    gsems = [g0, g1, g2, g3, g4, g5, g6, g7, g8, g9]
    ssems = [s0, s1, s2, s3, s4, s5, s6, s7, s8, s9]
    c = lax.axis_index("c")
    s = lax.axis_index("s")
    rows = pl.ds(s * RPS, RPS)
    pltpu.sync_copy(z80_hbm, asp.at[rows])
    plsc.subcore_barrier()

    def group(g, carry):
        c0 = g * NBUF2
        pltpu.sync_copy(src_hbm.at[s, pl.ds(c0, NBUF2)], srcg)
        pltpu.sync_copy(dst_hbm.at[s, pl.ds(c0, NBUF2)], dstg)
        cps = []
        for bi in range(NBUF2):
            cps.append(pltpu.async_copy(
                hs_hbm.at[c].at[srcg.at[bi]], rowbs[bi], gsems[bi]))
        scs = []
        for bi in range(NBUF2):
            cps[bi].wait()
            scs.append(pltpu.async_copy(
                rowbs[bi], asp.at[dstg.at[bi]], ssems[bi], add=True))
        for cp in scs:
            cp.wait()
        return carry

    lax.fori_loop(0, CH2 // NBUF2, group, 0)
    plsc.subcore_barrier()
    pltpu.sync_copy(asp.at[rows], a_out.at[c, rows])


@functools.lru_cache(maxsize=None)
def _pass2_kernel():
    return pl.kernel(
        _pass2_body,
        out_type=jax.ShapeDtypeStruct((2, N2, HHID), jnp.float32),
        mesh=_sc_mesh(),
        compiler_params=pltpu.CompilerParams(use_tc_tiling_on_sc=False,
                                             needs_layout_passes=False),
        scratch_types=(
            [pltpu.VMEM((NBUF2, K2), jnp.int32)] * 2
            + [pltpu.VMEM((K2, HHID), jnp.float32)] * NBUF2
            + [pltpu.VMEM_SHARED((N2, HHID), jnp.float32)]
            + [pltpu.SemaphoreType.DMA] * (2 * NBUF2)
        ),
    )


def _r5mat():
    lanes = lax.broadcasted_iota(jnp.int32, (16, HHID), 1) // OUT
    rowi = lax.broadcasted_iota(jnp.int32, (16, HHID), 0)
    return (lanes == rowi).astype(jnp.float32)


def _e15mat():
    rowi = lax.broadcasted_iota(jnp.int32, (16, HID), 0)
    return (rowi == 15).astype(jnp.float32)


def _halfsplit():
    rowi = lax.broadcasted_iota(jnp.int32, (HID, HHID), 0)
    coli = lax.broadcasted_iota(jnp.int32, (HID, HHID), 1)
    p0t = (rowi == coli).astype(jnp.float32)
    p1t = (rowi == coli + HHID).astype(jnp.float32)
    return p0t, p1t


def _halfcat():
    rowi = lax.broadcasted_iota(jnp.int32, (HHID, HID), 0)
    coli = lax.broadcasted_iota(jnp.int32, (HHID, HID), 1)
    p0 = (coli == rowi).astype(jnp.float32)
    p1 = (coli == rowi + HHID).astype(jnp.float32)
    return p0, p1


def _stage_c_body(u0_ref, u1_ref, dn0_ref, dn1_ref, gatb_ref, gw_ref, gcnb_ref,
                  hs_ref, base_ref):
    r5 = _r5mat()
    denb0 = jnp.dot(dn0_ref[0], r5, preferred_element_type=jnp.float32)
    denb1 = jnp.dot(dn1_ref[0], r5, preferred_element_type=jnp.float32)
    a0 = u0_ref[0] / (denb0 + 1e-16)
    a1 = u1_ref[0] / (denb1 + 1e-16)
    p0, p1 = _halfcat()
    agg = (jnp.dot(a0, p0, preferred_element_type=jnp.float32)
           + jnp.dot(a1, p1, preferred_element_type=jnp.float32))
    degb = jnp.dot(dn0_ref[0], _e15mat(), preferred_element_type=jnp.float32) + 1.0
    h1 = agg + gatb_ref[...]
    h1 = jnp.where(h1 > 0, h1, jnp.exp(h1) - 1.0)
    dinv = lax.rsqrt(jnp.maximum(degb, 1.0))
    h = jnp.dot(h1, gw_ref[...], preferred_element_type=jnp.float32)
    hsv = dinv * h
    p0t, p1t = _halfsplit()
    hs_ref[0] = jnp.dot(hsv, p0t, preferred_element_type=jnp.float32)
    hs_ref[1] = jnp.dot(hsv, p1t, preferred_element_type=jnp.float32)
    base_ref[...] = dinv * dinv * h + gcnb_ref[...]


def _stage_c(u0, u1, dn0, dn1, gatb, gw, gcnb):
    return pl.pallas_call(
        _stage_c_body,
        grid=(NBLK,),
        in_specs=[
            pl.BlockSpec((1, RB, HHID), lambda i: (0, i, 0)),
            pl.BlockSpec((1, RB, HHID), lambda i: (1, i, 0)),
            pl.BlockSpec((1, RB, 16), lambda i: (0, i, 0)),
            pl.BlockSpec((1, RB, 16), lambda i: (1, i, 0)),
            pl.BlockSpec((1, HID), lambda i: (0, 0)),
            pl.BlockSpec((HID, HID), lambda i: (0, 0)),
            pl.BlockSpec((1, HID), lambda i: (0, 0)),
        ],
        out_specs=[
            pl.BlockSpec((2, RB, HHID), lambda i: (0, i, 0)),
            pl.BlockSpec((RB, HID), lambda i: (i, 0)),
        ],
        out_shape=[
            jax.ShapeDtypeStruct((2, N, HHID), jnp.float32),
            jax.ShapeDtypeStruct((N, HID), jnp.float32),
        ],
    )(u0, u1, dn0, dn1, gatb, gw, gcnb)


def _stage_e_body(a0_ref, a1_ref, dn0_ref, base_ref, batch_ref,
                  out_ref, smax, ssum, scnt):
    i = pl.program_id(0)

    @pl.when(i == 0)
    def _init():
        smax[...] = jnp.zeros_like(smax)
        ssum[...] = jnp.zeros_like(ssum)
        scnt[...] = jnp.zeros_like(scnt)

    degb = jnp.dot(dn0_ref[0], _e15mat(),
                   preferred_element_type=jnp.float32) + 1.0
    dinv = lax.rsqrt(jnp.maximum(degb, 1.0))
    p0, p1 = _halfcat()
    acat = (jnp.dot(a0_ref[0], p0, preferred_element_type=jnp.float32)
            + jnp.dot(a1_ref[0], p1, preferred_element_type=jnp.float32))
    h2 = jnp.maximum(dinv * acat + base_ref[...], 0.0)

    b = batch_ref[...]
    onehot = (b == lax.broadcasted_iota(jnp.int32, (RB, G), 1)
              ).astype(jnp.float32)

    runmax = h2
    sft = 1
    while sft < RB:
        prev = jnp.concatenate(
            [jnp.zeros((sft, HID), jnp.float32), runmax[:-sft]], axis=0)
        bprev = jnp.concatenate(
            [jnp.full((sft, 1), -1, jnp.int32), b[:-sft]], axis=0)
        same = (b == bprev)
        runmax = jnp.maximum(runmax, jnp.where(same, prev, 0.0))
        sft *= 2
    bnxt = jnp.concatenate([b[1:], jnp.full((1, 1), -1, jnp.int32)], axis=0)
    lastm = (b != bnxt).astype(jnp.float32)
    ext = runmax * lastm
    blkmax = lax.dot_general(onehot, ext, (((0,), (0,)), ((), ())),
                             preferred_element_type=jnp.float32)
    smax[...] = jnp.maximum(smax[...], blkmax)
    ssum[...] += lax.dot_general(onehot, h2, (((0,), (0,)), ((), ())),
                                 preferred_element_type=jnp.float32)
    scnt[...] += lax.dot_general(onehot, jnp.ones((RB, 128), jnp.float32),
                                 (((0,), (0,)), ((), ())),
                                 preferred_element_type=jnp.float32)

    @pl.when(i == NBLK - 1)
    def _fin():
        cnt = jnp.maximum(scnt[:, :1], 1.0)
        out_ref[:, :HID] = smax[...]
        out_ref[:, HID:] = ssum[...] / cnt


def _stage_e(a0, a1, dn0, base, batchr):
    return pl.pallas_call(
        _stage_e_body,
        grid=(NBLK,),
        in_specs=[
            pl.BlockSpec((1, RB, HHID), lambda i: (0, i, 0)),
            pl.BlockSpec((1, RB, HHID), lambda i: (1, i, 0)),
            pl.BlockSpec((1, RB, 16), lambda i: (0, i, 0)),
            pl.BlockSpec((RB, HID), lambda i: (i, 0)),
            pl.BlockSpec((RB, 1), lambda i: (i, 0)),
        ],
        out_specs=pl.BlockSpec((G, 2 * HID), lambda i: (0, 0)),
        out_shape=jax.ShapeDtypeStruct((G, 2 * HID), jnp.float32),
        scratch_shapes=[
            pltpu.VMEM((G, HID), jnp.float32),
            pltpu.VMEM((G, HID), jnp.float32),
            pltpu.VMEM((G, 128), jnp.float32),
        ],
    )(a0, a1, dn0, base, batchr)


def kernel(x, edge_index, batch, gat_Wl, gat_Wr, gat_att, gat_bias, gcn_W, gcn_bias):
    src = edge_index[0]
    dst = edge_index[1]
    xl2, xr2 = _mm2(x, gat_Wl[:, :HHID], gat_Wl[:, HHID:],
                    gat_Wr[:, :HHID], gat_Wr[:, HHID:])
    att2 = gat_att.reshape(2, HHID)
    z80 = jnp.zeros((RPS, HHID), jnp.float32)
    z16 = jnp.zeros((RPS, 16), jnp.float32)
    u2, den2 = _pass1_kernel()(xl2, xr2, src.reshape(16, CH1, K1),
                               dst.reshape(16, CH1, K1), att2, z80, z16)
    hs, base = _stage_c(u2, u2, den2, den2,
                        gat_bias.reshape(1, HID), gcn_W, gcn_bias.reshape(1, HID))
    a2 = _pass2_kernel()(hs, src.reshape(16, CH2, K2), dst.reshape(16, CH2, K2), z80)
    batchr = batch.reshape(N, 1)
    return _stage_e(a2, a2, den2, base, batchr)


# speedup vs baseline: 1.0302x; 1.0302x over previous
import functools

import jax
import jax.numpy as jnp
from jax import lax
from jax.experimental import pallas as pl
from jax.experimental.pallas import tpu as pltpu
from jax.experimental.pallas import tpu_sc as plsc

N = 10000
E = 320000
D = 128
OUT = 16
HEADS = 10
HID = OUT * HEADS
G = 64

RB = 1000
NBLK = N // RB
NW = 32
EPW = E // NW
HHID = HID // 2
EPW1 = E // 16
K1 = 40
CH1 = EPW1 // K1
K2 = 40
CH2 = EPW1 // K2
N2 = 10240
RPS = N2 // 16


def _mm2_body(x_ref, wla_ref, wlb_ref, wra_ref, wrb_ref, xl_ref, xr_ref):
    xb = x_ref[...]
    xl_ref[0] = jnp.dot(xb, wla_ref[...], preferred_element_type=jnp.float32)
    xl_ref[1] = jnp.dot(xb, wlb_ref[...], preferred_element_type=jnp.float32)
    xr_ref[0] = jnp.dot(xb, wra_ref[...], preferred_element_type=jnp.float32)
    xr_ref[1] = jnp.dot(xb, wrb_ref[...], preferred_element_type=jnp.float32)


def _mm2(x, wla, wlb, wra, wrb):
    wspec = pl.BlockSpec((D, HHID), lambda i: (0, 0))
    ospec = pl.BlockSpec((2, RB, HHID), lambda i: (0, i, 0))
    return pl.pallas_call(
        _mm2_body,
        grid=(NBLK,),
        in_specs=[pl.BlockSpec((RB, D), lambda i: (i, 0)),
                  wspec, wspec, wspec, wspec],
        out_specs=[ospec, ospec],
        out_shape=[
            jax.ShapeDtypeStruct((2, N, HHID), jnp.float32),
            jax.ShapeDtypeStruct((2, N, HHID), jnp.float32),
        ],
    )(x, wla, wlb, wra, wrb)


def _sc_mesh():
    return plsc.VectorSubcoreMesh(core_axis_name="c", subcore_axis_name="s",
                                  num_cores=2, num_subcores=16)


NBUF1 = 5
NBUF2 = 10


def _pass1_body(xl_hbm, xr_hbm, src_hbm, dst_hbm, att_hbm, z80_hbm, z16_hbm,
                u_out, den_out,
                srcg, dstg,
                xlb0, xlb1, xlb2, xlb3, xlb4,
                xrb0, xrb1, xrb2, xrb3, xrb4,
                wb0, wb1, wb2, wb3, wb4, attb, usp, densp,
                g0, g1, g2, g3, g4, s0, s1, s2, s3, s4):
    xlbs = [xlb0, xlb1, xlb2, xlb3, xlb4]
    xrbs = [xrb0, xrb1, xrb2, xrb3, xrb4]
    wbs = [wb0, wb1, wb2, wb3, wb4]
    gsems = [g0, g1, g2, g3, g4]
    ssems = [s0, s1, s2, s3, s4]
    c = lax.axis_index("c")
    s = lax.axis_index("s")
    rows = pl.ds(s * RPS, RPS)
    pltpu.sync_copy(z80_hbm, usp.at[rows])
    pltpu.sync_copy(z16_hbm, densp.at[rows])
    pltpu.sync_copy(att_hbm.at[c], attb)
    plsc.subcore_barrier()
    attvs = [attb[pl.ds(16 * h, 16)] for h in range(HEADS // 2)]

    def group(g, carry):
        c0 = g * NBUF1
        pltpu.sync_copy(src_hbm.at[s, pl.ds(c0, NBUF1)], srcg)
        pltpu.sync_copy(dst_hbm.at[s, pl.ds(c0, NBUF1)], dstg)
        cps = []
        for bi in range(NBUF1):
            cps.append(pltpu.async_copy(
                xl_hbm.at[c].at[srcg.at[bi]], xlbs[bi], gsems[bi]))
            cps.append(pltpu.async_copy(
                xr_hbm.at[c].at[dstg.at[bi]], xrbs[bi], gsems[bi]))
        scs = []
        for bi in range(NBUF1):
            xlb, xrb, wb = xlbs[bi], xrbs[bi], wbs[bi]
            cps[2 * bi].wait()
            cps[2 * bi + 1].wait()

            @plsc.parallel_loop(0, K1, unroll=2)
            def edge(e):
                lanei = lax.iota(jnp.int32, 16)
                xh = [xlb[e, pl.ds(OUT * h, OUT)] for h in range(HEADS // 2)]
                logitv = jnp.zeros((16,), jnp.float32)
                for h in range(HEADS // 2):
                    z = xh[h] + xrb[e, pl.ds(OUT * h, OUT)]
                    lz = jnp.maximum(z, 0.2 * z)
                    sh = jnp.sum(lz * attvs[h])
                    logitv = jnp.where(lanei == h, sh, logitv)
                w = jnp.exp(logitv)
                wb[e, :] = w
                for h in range(HEADS // 2):
                    xlb[e, pl.ds(OUT * h, OUT)] = xh[h] * w[h]
            scs.append(pltpu.async_copy(
                xlb, usp.at[dstg.at[bi]], gsems[bi], add=True))
            scs.append(pltpu.async_copy(
                wb, densp.at[dstg.at[bi]], ssems[bi], add=True))
        for cp in scs:
            cp.wait()
        return carry

    lax.fori_loop(0, CH1 // NBUF1, group, 0)
    plsc.subcore_barrier()
    pltpu.sync_copy(usp.at[rows], u_out.at[c, rows])
    pltpu.sync_copy(densp.at[rows], den_out.at[c, rows])


@functools.lru_cache(maxsize=None)
def _pass1_kernel():
    return pl.kernel(
        _pass1_body,
        out_type=[
            jax.ShapeDtypeStruct((2, N2, HHID), jnp.float32),
            jax.ShapeDtypeStruct((2, N2, 16), jnp.float32),
        ],
        mesh=_sc_mesh(),
        compiler_params=pltpu.CompilerParams(use_tc_tiling_on_sc=False,
                                             needs_layout_passes=False),
        scratch_types=(
            [pltpu.VMEM((NBUF1, K1), jnp.int32)] * 2
            + [pltpu.VMEM((K1, HHID), jnp.float32)] * (2 * NBUF1)
            + [pltpu.VMEM((K1, 16), jnp.float32)] * NBUF1
            + [pltpu.VMEM((HHID,), jnp.float32)]
            + [pltpu.VMEM_SHARED((N2, HHID), jnp.float32),
               pltpu.VMEM_SHARED((N2, 16), jnp.float32)]
            + [pltpu.SemaphoreType.DMA] * (2 * NBUF1)
        ),
    )


def _pass2_body(hs_hbm, src_hbm, dst_hbm, z80_hbm,
                a_out,
                srcg, dstg, rb0, rb1, rb2, rb3, rb4, rb5, rb6, rb7, rb8, rb9,
                asp,
                g0, g1, g2, g3, g4, g5, g6, g7, g8, g9,
                s0, s1, s2, s3, s4, s5, s6, s7, s8, s9):
    rowbs = [rb0, rb1, rb2, rb3, rb4, rb5, rb6, rb7, rb8, rb9]
    gsems = [g0, g1, g2, g3, g4, g5, g6, g7, g8, g9]
    ssems = [s0, s1, s2, s3, s4, s5, s6, s7, s8, s9]
    c = lax.axis_index("c")
    s = lax.axis_index("s")
    rows = pl.ds(s * RPS, RPS)
    pltpu.sync_copy(z80_hbm, asp.at[rows])
    plsc.subcore_barrier()

    def group(g, carry):
        c0 = g * NBUF2
        pltpu.sync_copy(src_hbm.at[s, pl.ds(c0, NBUF2)], srcg)
        pltpu.sync_copy(dst_hbm.at[s, pl.ds(c0, NBUF2)], dstg)
        cps = []
        for bi in range(NBUF2):
            cps.append(pltpu.async_copy(
                hs_hbm.at[c].at[srcg.at[bi]], rowbs[bi], gsems[bi]))
        scs = []
        for bi in range(NBUF2):
            cps[bi].wait()
            scs.append(pltpu.async_copy(
                rowbs[bi], asp.at[dstg.at[bi]], ssems[bi], add=True))
        for cp in scs:
            cp.wait()
        return carry

    lax.fori_loop(0, CH2 // NBUF2, group, 0)
    plsc.subcore_barrier()
    pltpu.sync_copy(asp.at[rows], a_out.at[c, rows])


@functools.lru_cache(maxsize=None)
def _pass2_kernel():
    return pl.kernel(
        _pass2_body,
        out_type=jax.ShapeDtypeStruct((2, N2, HHID), jnp.float32),
        mesh=_sc_mesh(),
        compiler_params=pltpu.CompilerParams(use_tc_tiling_on_sc=False,
                                             needs_layout_passes=False),
        scratch_types=(
            [pltpu.VMEM((NBUF2, K2), jnp.int32)] * 2
            + [pltpu.VMEM((K2, HHID), jnp.float32)] * NBUF2
            + [pltpu.VMEM_SHARED((N2, HHID), jnp.float32)]
            + [pltpu.SemaphoreType.DMA] * (2 * NBUF2)
        ),
    )


def _r5mat():
    lanes = lax.broadcasted_iota(jnp.int32, (16, HHID), 1) // OUT
    rowi = lax.broadcasted_iota(jnp.int32, (16, HHID), 0)
    return (lanes == rowi).astype(jnp.float32)


def _e15mat():
    rowi = lax.broadcasted_iota(jnp.int32, (16, HID), 0)
    return (rowi == 15).astype(jnp.float32)


def _halfsplit():
    rowi = lax.broadcasted_iota(jnp.int32, (HID, HHID), 0)
    coli = lax.broadcasted_iota(jnp.int32, (HID, HHID), 1)
    p0t = (rowi == coli).astype(jnp.float32)
    p1t = (rowi == coli + HHID).astype(jnp.float32)
    return p0t, p1t


def _halfcat():
    rowi = lax.broadcasted_iota(jnp.int32, (HHID, HID), 0)
    coli = lax.broadcasted_iota(jnp.int32, (HHID, HID), 1)
    p0 = (coli == rowi).astype(jnp.float32)
    p1 = (coli == rowi + HHID).astype(jnp.float32)
    return p0, p1


def _stage_c_body(u0_ref, u1_ref, dn0_ref, dn1_ref, gatb_ref, gw_ref, gcnb_ref,
                  hs_ref, base_ref):
    r5 = _r5mat()
    denb0 = jnp.dot(dn0_ref[0], r5, preferred_element_type=jnp.float32)
    denb1 = jnp.dot(dn1_ref[0], r5, preferred_element_type=jnp.float32)
    a0 = u0_ref[0] / (denb0 + 1e-16)
    a1 = u1_ref[0] / (denb1 + 1e-16)
    p0, p1 = _halfcat()
    agg = (jnp.dot(a0, p0, preferred_element_type=jnp.float32)
           + jnp.dot(a1, p1, preferred_element_type=jnp.float32))
    degb = jnp.dot(dn0_ref[0], _e15mat(), preferred_element_type=jnp.float32) + 1.0
    h1 = agg + gatb_ref[...]
    h1 = jnp.where(h1 > 0, h1, jnp.exp(h1) - 1.0)
    dinv = lax.rsqrt(jnp.maximum(degb, 1.0))
    h = jnp.dot(h1, gw_ref[...], preferred_element_type=jnp.float32)
    hsv = dinv * h
    p0t, p1t = _halfsplit()
    hs_ref[0] = jnp.dot(hsv, p0t, preferred_element_type=jnp.float32)
    hs_ref[1] = jnp.dot(hsv, p1t, preferred_element_type=jnp.float32)
    base_ref[...] = dinv * dinv * h + gcnb_ref[...]


def _stage_c(u0, u1, dn0, dn1, gatb, gw, gcnb):
    return pl.pallas_call(
        _stage_c_body,
        grid=(NBLK,),
        in_specs=[
            pl.BlockSpec((1, RB, HHID), lambda i: (0, i, 0)),
            pl.BlockSpec((1, RB, HHID), lambda i: (1, i, 0)),
            pl.BlockSpec((1, RB, 16), lambda i: (0, i, 0)),
            pl.BlockSpec((1, RB, 16), lambda i: (1, i, 0)),
            pl.BlockSpec((1, HID), lambda i: (0, 0)),
            pl.BlockSpec((HID, HID), lambda i: (0, 0)),
            pl.BlockSpec((1, HID), lambda i: (0, 0)),
        ],
        out_specs=[
            pl.BlockSpec((2, RB, HHID), lambda i: (0, i, 0)),
            pl.BlockSpec((RB, HID), lambda i: (i, 0)),
        ],
        out_shape=[
            jax.ShapeDtypeStruct((2, N, HHID), jnp.float32),
            jax.ShapeDtypeStruct((N, HID), jnp.float32),
        ],
    )(u0, u1, dn0, dn1, gatb, gw, gcnb)


def _stage_e_body(a0_ref, a1_ref, dn0_ref, base_ref, batch_ref,
                  out_ref, smax, ssum, scnt):
    i = pl.program_id(0)

    @pl.when(i == 0)
    def _init():
        smax[...] = jnp.zeros_like(smax)
        ssum[...] = jnp.zeros_like(ssum)
        scnt[...] = jnp.zeros_like(scnt)

    degb = jnp.dot(dn0_ref[0], _e15mat(),
                   preferred_element_type=jnp.float32) + 1.0
    dinv = lax.rsqrt(jnp.maximum(degb, 1.0))
    p0, p1 = _halfcat()
    acat = (jnp.dot(a0_ref[0], p0, preferred_element_type=jnp.float32)
            + jnp.dot(a1_ref[0], p1, preferred_element_type=jnp.float32))
    h2 = jnp.maximum(dinv * acat + base_ref[...], 0.0)

    b = batch_ref[...]
    onehot = (b == lax.broadcasted_iota(jnp.int32, (RB, G), 1)
              ).astype(jnp.float32)

    runmax = h2
    sft = 1
    while sft < RB:
        prev = jnp.concatenate(
            [jnp.zeros((sft, HID), jnp.float32), runmax[:-sft]], axis=0)
        bprev = jnp.concatenate(
            [jnp.full((sft, 1), -1, jnp.int32), b[:-sft]], axis=0)
        same = (b == bprev)
        runmax = jnp.maximum(runmax, jnp.where(same, prev, 0.0))
        sft *= 2
    bnxt = jnp.concatenate([b[1:], jnp.full((1, 1), -1, jnp.int32)], axis=0)
    lastm = (b != bnxt).astype(jnp.float32)
    ext = runmax * lastm
    blkmax = lax.dot_general(onehot, ext, (((0,), (0,)), ((), ())),
                             preferred_element_type=jnp.float32)
    smax[...] = jnp.maximum(smax[...], blkmax)
    ssum[...] += lax.dot_general(onehot, h2, (((0,), (0,)), ((), ())),
                                 preferred_element_type=jnp.float32)
    scnt[...] += lax.dot_general(onehot, jnp.ones((RB, 128), jnp.float32),
                                 (((0,), (0,)), ((), ())),
                                 preferred_element_type=jnp.float32)

    @pl.when(i == NBLK - 1)
    def _fin():
        cnt = jnp.maximum(scnt[:, :1], 1.0)
        out_ref[:, :HID] = smax[...]
        out_ref[:, HID:] = ssum[...] / cnt


def _stage_e(a0, a1, dn0, base, batchr):
    return pl.pallas_call(
        _stage_e_body,
        grid=(NBLK,),
        in_specs=[
            pl.BlockSpec((1, RB, HHID), lambda i: (0, i, 0)),
            pl.BlockSpec((1, RB, HHID), lambda i: (1, i, 0)),
            pl.BlockSpec((1, RB, 16), lambda i: (0, i, 0)),
            pl.BlockSpec((RB, HID), lambda i: (i, 0)),
            pl.BlockSpec((RB, 1), lambda i: (i, 0)),
        ],
        out_specs=pl.BlockSpec((G, 2 * HID), lambda i: (0, 0)),
        out_shape=jax.ShapeDtypeStruct((G, 2 * HID), jnp.float32),
        scratch_shapes=[
            pltpu.VMEM((G, HID), jnp.float32),
            pltpu.VMEM((G, HID), jnp.float32),
            pltpu.VMEM((G, 128), jnp.float32),
        ],
    )(a0, a1, dn0, base, batchr)


def kernel(x, edge_index, batch, gat_Wl, gat_Wr, gat_att, gat_bias, gcn_W, gcn_bias):
    src = edge_index[0]
    dst = edge_index[1]
    xl2, xr2 = _mm2(x, gat_Wl[:, :HHID], gat_Wl[:, HHID:],
                    gat_Wr[:, :HHID], gat_Wr[:, HHID:])
    att2 = gat_att.reshape(2, HHID)
    z80 = jnp.zeros((RPS, HHID), jnp.float32)
    z16 = jnp.zeros((RPS, 16), jnp.float32)
    u2, den2 = _pass1_kernel()(xl2, xr2, src.reshape(16, CH1, K1),
                               dst.reshape(16, CH1, K1), att2, z80, z16)
    hs, base = _stage_c(u2, u2, den2, den2,
                        gat_bias.reshape(1, HID), gcn_W, gcn_bias.reshape(1, HID))
    a2 = _pass2_kernel()(hs, src.reshape(16, CH2, K2), dst.reshape(16, CH2, K2), z80)
    batchr = batch.reshape(N, 1)
    return _stage_e(a2, a2, den2, base, batchr)
